# Initial kernel scaffold; baseline (speedup 1.0000x reference)
import jax
import jax.numpy as jnp
from jax.experimental import pallas as pl


def kernel(input_tensor, indexes):
    raise NotImplementedError("write your pallas kernel here")


# speedup vs baseline: 2.9690x; 2.9690x over previous
import jax
import jax.numpy as jnp
from jax.experimental import pallas as pl
from jax.experimental.pallas import tpu as pltpu


def _sel_kernel(mask_ref, sel_ref, nsel_ref):
    c = mask_ref.shape[-1]
    nz = mask_ref[...] != 0.0
    nzi = nz.astype(jnp.int32)
    row = jax.lax.broadcasted_iota(jnp.int32, (c, c), 0)
    col = jax.lax.broadcasted_iota(jnp.int32, (c, c), 1)
    rank = jnp.sum((nz & (col < row)).astype(jnp.int32), axis=1)
    m = nz & (jnp.broadcast_to(rank[None, :], (c, c)) == row)
    sel = jnp.sum(jnp.where(m, col, 0), axis=1)
    sel_ref[...] = sel.reshape(1, c)
    nsel_ref[...] = jnp.sum(nzi, axis=-1, keepdims=True)


def _copy_kernel(sel_ref, nsel_ref, in_ref, out_ref):
    del sel_ref
    j = pl.program_id(0)

    @pl.when(j < nsel_ref[0])
    def _valid():
        out_ref[...] = in_ref[...]

    @pl.when(j >= nsel_ref[0])
    def _invalid():
        out_ref[...] = jnp.full_like(out_ref, jnp.nan)


def kernel(input_tensor, indexes):
    n, c, h, w = input_tensor.shape

    sel, nsel = pl.pallas_call(
        _sel_kernel,
        out_shape=(
            jax.ShapeDtypeStruct((1, c), jnp.int32),
            jax.ShapeDtypeStruct((1, 1), jnp.int32),
        ),
    )(indexes.reshape(1, c))
    sel = sel.reshape(c)
    nsel = nsel.reshape(1)

    grid_spec = pltpu.PrefetchScalarGridSpec(
        num_scalar_prefetch=2,
        grid=(c,),
        in_specs=[
            pl.BlockSpec(
                (n, 1, h, w), lambda j, sel_ref, nsel_ref: (0, sel_ref[j], 0, 0)
            )
        ],
        out_specs=pl.BlockSpec(
            (n, 1, h, w), lambda j, sel_ref, nsel_ref: (0, j, 0, 0)
        ),
    )
    return pl.pallas_call(
        _copy_kernel,
        grid_spec=grid_spec,
        out_shape=jax.ShapeDtypeStruct((n, c, h, w), input_tensor.dtype),
    )(sel, nsel, input_tensor)
